# baseline (device time: 82368 ns/iter reference)
import jax
import jax.numpy as jnp
from jax import lax
from jax.experimental import pallas as pl
from jax.experimental.pallas import tpu as pltpu

N_DEV = 8
SQ = 512
D = 1024
DH = 128
HQ_LOC = 8
KV_LOC = 2
CHUNK = SQ // N_DEV
SCALE = 0.08838834764831843


def kernel(x, Wq, Wo, Wk, Wv):
    my = lax.axis_index("i")
    x2 = x.reshape(SQ, D)
    wk_loc = lax.dynamic_slice(Wk, (0, my * (KV_LOC * DH)), (D, KV_LOC * DH))
    wv_loc = lax.dynamic_slice(Wv, (0, my * (KV_LOC * DH)), (D, KV_LOC * DH))

    def body(x_ref, wq_ref, wo_ref, wk_ref, wv_ref, out_ref,
             p_ref, rs_ref, ag_ref,
             rs_send_sems, rs_recv_sems, ag_send_sems, ag_recv_sems):
        my_i = lax.axis_index("i")
        left = (my_i + N_DEV - 1) % N_DEV
        right = (my_i + 1) % N_DEV

        barrier_sem = pltpu.get_barrier_semaphore()
        for nbr in (left, right):
            pl.semaphore_signal(
                barrier_sem, inc=1,
                device_id=(nbr,), device_id_type=pl.DeviceIdType.MESH,
            )
        pl.semaphore_wait(barrier_sem, 2)

        xv = x_ref[...]
        q = jnp.dot(xv, wq_ref[...], preferred_element_type=jnp.float32)
        k = jnp.dot(xv, wk_ref[...], preferred_element_type=jnp.float32)
        v = jnp.dot(xv, wv_ref[...], preferred_element_type=jnp.float32)

        o_heads = []
        for h in range(HQ_LOC):
            qh = q[:, h * DH:(h + 1) * DH]
            kv = h // 4
            kh = k[:, kv * DH:(kv + 1) * DH]
            vh = v[:, kv * DH:(kv + 1) * DH]
            s = jnp.dot(qh, kh.T, preferred_element_type=jnp.float32) * SCALE
            m = jnp.max(s, axis=-1, keepdims=True)
            p = jnp.exp(s - m)
            l = jnp.sum(p, axis=-1, keepdims=True)
            o_heads.append(jnp.dot(p, vh, preferred_element_type=jnp.float32) / l)
        o_loc = jnp.concatenate(o_heads, axis=1)

        p_ref[...] = jnp.dot(o_loc, wo_ref[...], preferred_element_type=jnp.float32)

        rs_ref[0, :, :] = p_ref[pl.ds(my_i * CHUNK, CHUNK), :]
        for s in range(N_DEV - 1):
            rdma = pltpu.make_async_remote_copy(
                src_ref=rs_ref.at[s],
                dst_ref=rs_ref.at[s + 1],
                send_sem=rs_send_sems.at[s],
                recv_sem=rs_recv_sems.at[s],
                device_id=(right,),
                device_id_type=pl.DeviceIdType.MESH,
            )
            rdma.start()
            rdma.wait()
            c = (my_i + (N_DEV - s - 1)) % N_DEV
            rs_ref[s + 1, :, :] = rs_ref[s + 1, :, :] + p_ref[pl.ds(c * CHUNK, CHUNK), :]

        own_chunk = (my_i + 1) % N_DEV
        out_ref[0, pl.ds(own_chunk * CHUNK, CHUNK), :] = rs_ref[N_DEV - 1, :, :]

        ag_ref[0, :, :] = rs_ref[N_DEV - 1, :, :]
        for s in range(N_DEV - 1):
            rdma = pltpu.make_async_remote_copy(
                src_ref=ag_ref.at[s],
                dst_ref=ag_ref.at[s + 1],
                send_sem=ag_send_sems.at[s],
                recv_sem=ag_recv_sems.at[s],
                device_id=(right,),
                device_id_type=pl.DeviceIdType.MESH,
            )
            rdma.start()
            rdma.wait()
            c = (my_i + (N_DEV - s)) % N_DEV
            out_ref[0, pl.ds(c * CHUNK, CHUNK), :] = ag_ref[s + 1, :, :]

    out = pl.pallas_call(
        body,
        out_shape=jax.ShapeDtypeStruct((1, SQ, D), jnp.float32),
        in_specs=[
            pl.BlockSpec(memory_space=pltpu.VMEM),
            pl.BlockSpec(memory_space=pltpu.VMEM),
            pl.BlockSpec(memory_space=pltpu.VMEM),
            pl.BlockSpec(memory_space=pltpu.VMEM),
            pl.BlockSpec(memory_space=pltpu.VMEM),
        ],
        out_specs=pl.BlockSpec(memory_space=pltpu.VMEM),
        scratch_shapes=[
            pltpu.VMEM((SQ, D), jnp.float32),
            pltpu.VMEM((N_DEV, CHUNK, D), jnp.float32),
            pltpu.VMEM((N_DEV, CHUNK, D), jnp.float32),
            pltpu.SemaphoreType.DMA((N_DEV - 1,)),
            pltpu.SemaphoreType.DMA((N_DEV - 1,)),
            pltpu.SemaphoreType.DMA((N_DEV - 1,)),
            pltpu.SemaphoreType.DMA((N_DEV - 1,)),
        ],
        compiler_params=pltpu.CompilerParams(collective_id=0),
    )(x2, Wq, Wo, wk_loc, wv_loc)
    return out


# device time: 67079 ns/iter; 1.2279x vs baseline; 1.2279x over previous
import jax
import jax.numpy as jnp
from jax import lax
from jax.experimental import pallas as pl
from jax.experimental.pallas import tpu as pltpu

N_DEV = 8
SQ = 512
D = 1024
DH = 128
HQ_LOC = 8
KV_LOC = 2
SCALE = 0.08838834764831843

_STAGES = ((1, 256), (3, 128), (4, 64))


def kernel(x, Wq, Wo, Wk, Wv):
    my = lax.axis_index("i")
    x2 = x.reshape(SQ, D)
    wk_loc = lax.dynamic_slice(Wk, (0, my * (KV_LOC * DH)), (D, KV_LOC * DH))
    wv_loc = lax.dynamic_slice(Wv, (0, my * (KV_LOC * DH)), (D, KV_LOC * DH))

    def body(x_ref, wq_ref, wo_ref, wk_ref, wv_ref, out_ref,
             p_ref, rs0_ref, rs1_ref, rs2_ref, send_sems, recv_sems):
        my_i = lax.axis_index("i")
        q4 = my_i % 4
        bz = my_i // 4
        by = q4 // 2
        bx = (q4 % 2) ^ by
        bits = (bx, by, bz)
        partners = tuple((my_i ^ m) for m, _ in _STAGES)

        barrier_sem = pltpu.get_barrier_semaphore()
        for nbr in partners:
            pl.semaphore_signal(
                barrier_sem, inc=1,
                device_id=(nbr,), device_id_type=pl.DeviceIdType.MESH,
            )
        pl.semaphore_wait(barrier_sem, 3)

        xv = x_ref[...]
        qm = jnp.dot(xv, wq_ref[...], preferred_element_type=jnp.float32)
        km = jnp.dot(xv, wk_ref[...], preferred_element_type=jnp.float32)
        vm = jnp.dot(xv, wv_ref[...], preferred_element_type=jnp.float32)

        o_heads = []
        for h in range(HQ_LOC):
            qh = qm[:, h * DH:(h + 1) * DH]
            kv = h // 4
            kh = km[:, kv * DH:(kv + 1) * DH]
            vh = vm[:, kv * DH:(kv + 1) * DH]
            s = jnp.dot(qh, kh.T, preferred_element_type=jnp.float32) * SCALE
            m = jnp.max(s, axis=-1, keepdims=True)
            p = jnp.exp(s - m)
            l = jnp.sum(p, axis=-1, keepdims=True)
            o_heads.append(jnp.dot(p, vh, preferred_element_type=jnp.float32) / l)
        o_loc = jnp.concatenate(o_heads, axis=1)

        p_ref[...] = jnp.dot(o_loc, wo_ref[...], preferred_element_type=jnp.float32)

        rs_bufs = (rs0_ref, rs1_ref, rs2_ref)
        off = 0
        for k, (mask, seg) in enumerate(_STAGES):
            b = bits[k]
            send_off = off + (1 - b) * seg
            keep_off = off + b * seg
            rdma = pltpu.make_async_remote_copy(
                src_ref=p_ref.at[pl.ds(send_off, seg), :],
                dst_ref=rs_bufs[k],
                send_sem=send_sems.at[k],
                recv_sem=recv_sems.at[k],
                device_id=(partners[k],),
                device_id_type=pl.DeviceIdType.MESH,
            )
            rdma.start()
            rdma.wait()
            p_ref[pl.ds(keep_off, seg), :] = (
                p_ref[pl.ds(keep_off, seg), :] + rs_bufs[k][...]
            )
            off = keep_off

        out_ref[0, pl.ds(off, 64), :] = p_ref[pl.ds(off, 64), :]

        for k in (2, 1, 0):
            mask, seg = _STAGES[k]
            b = bits[k]
            my_off = off
            rdma = pltpu.make_async_remote_copy(
                src_ref=out_ref.at[0, pl.ds(my_off, seg), :],
                dst_ref=out_ref.at[0, pl.ds(my_off, seg), :],
                send_sem=send_sems.at[3 + k],
                recv_sem=recv_sems.at[3 + k],
                device_id=(partners[k],),
                device_id_type=pl.DeviceIdType.MESH,
            )
            rdma.start()
            rdma.wait()
            off = my_off - b * seg

    out = pl.pallas_call(
        body,
        out_shape=jax.ShapeDtypeStruct((1, SQ, D), jnp.float32),
        in_specs=[pl.BlockSpec(memory_space=pltpu.VMEM)] * 5,
        out_specs=pl.BlockSpec(memory_space=pltpu.VMEM),
        scratch_shapes=[
            pltpu.VMEM((SQ, D), jnp.float32),
            pltpu.VMEM((256, D), jnp.float32),
            pltpu.VMEM((128, D), jnp.float32),
            pltpu.VMEM((64, D), jnp.float32),
            pltpu.SemaphoreType.DMA((6,)),
            pltpu.SemaphoreType.DMA((6,)),
        ],
        compiler_params=pltpu.CompilerParams(collective_id=0),
    )(x2, Wq, Wo, wk_loc, wv_loc)
    return out


# device time: 47455 ns/iter; 1.7357x vs baseline; 1.4135x over previous
import jax
import jax.numpy as jnp
from jax import lax
from jax.experimental import pallas as pl
from jax.experimental.pallas import tpu as pltpu

N_DEV = 8
SQ = 512
D = 1024
DH = 128
HQ_LOC = 8
KV_LOC = 2
SCALE = 0.08838834764831843

_STAGES = ((1, 256), (3, 128), (4, 64))


def kernel(x, Wq, Wo, Wk, Wv):
    my = lax.axis_index("i")
    x2 = x.reshape(SQ, D)
    wk_loc = lax.dynamic_slice(Wk, (0, my * (KV_LOC * DH)), (D, KV_LOC * DH))
    wv_loc = lax.dynamic_slice(Wv, (0, my * (KV_LOC * DH)), (D, KV_LOC * DH))

    def body(x_ref, wq_ref, wo_ref, wk_ref, wv_ref, out_ref,
             p_ref, ag_ref, rs0_ref, rs1_ref, rs2_ref, send_sems, recv_sems):
        my_i = lax.axis_index("i")
        q4 = my_i % 4
        bz = my_i // 4
        by = q4 // 2
        bx = (q4 % 2) ^ by
        bits = (bx, by, bz)
        partners = tuple((my_i ^ m) for m, _ in _STAGES)

        barrier_sem = pltpu.get_barrier_semaphore()
        for nbr in partners:
            pl.semaphore_signal(
                barrier_sem, inc=1,
                device_id=(nbr,), device_id_type=pl.DeviceIdType.MESH,
            )
        pl.semaphore_wait(barrier_sem, 3)

        xv = x_ref[...].astype(jnp.bfloat16)
        qm = jnp.dot(xv, wq_ref[...].astype(jnp.bfloat16),
                     preferred_element_type=jnp.float32)
        km = jnp.dot(xv, wk_ref[...].astype(jnp.bfloat16),
                     preferred_element_type=jnp.float32)
        vm = jnp.dot(xv, wv_ref[...].astype(jnp.bfloat16),
                     preferred_element_type=jnp.float32)

        o_heads = []
        for h in range(HQ_LOC):
            qh = qm[:, h * DH:(h + 1) * DH].astype(jnp.bfloat16)
            kv = h // 4
            kh = km[:, kv * DH:(kv + 1) * DH].astype(jnp.bfloat16)
            vh = vm[:, kv * DH:(kv + 1) * DH].astype(jnp.bfloat16)
            s = jnp.dot(qh, kh.T, preferred_element_type=jnp.float32) * SCALE
            m = jnp.max(s, axis=-1, keepdims=True)
            p = jnp.exp(s - m).astype(jnp.bfloat16)
            l = jnp.sum(p, axis=-1, keepdims=True, dtype=jnp.float32)
            o_heads.append(
                jnp.dot(p, vh, preferred_element_type=jnp.float32) / l
            )
        o_loc = jnp.concatenate(o_heads, axis=1).astype(jnp.bfloat16)

        p_ref[...] = jnp.dot(
            o_loc, wo_ref[...].astype(jnp.bfloat16),
            preferred_element_type=jnp.float32,
        ).astype(jnp.bfloat16)

        rs_bufs = (rs0_ref, rs1_ref, rs2_ref)
        off = 0
        for k, (mask, seg) in enumerate(_STAGES):
            b = bits[k]
            send_off = off + (1 - b) * seg
            keep_off = off + b * seg
            rdma = pltpu.make_async_remote_copy(
                src_ref=p_ref.at[pl.ds(send_off, seg), :],
                dst_ref=rs_bufs[k],
                send_sem=send_sems.at[k],
                recv_sem=recv_sems.at[k],
                device_id=(partners[k],),
                device_id_type=pl.DeviceIdType.MESH,
            )
            rdma.start()
            rdma.wait()
            p_ref[pl.ds(keep_off, seg), :] = (
                p_ref[pl.ds(keep_off, seg), :] + rs_bufs[k][...]
            )
            off = keep_off

        ag_ref[pl.ds(off, 64), :] = p_ref[pl.ds(off, 64), :]

        for k in (2, 1, 0):
            mask, seg = _STAGES[k]
            b = bits[k]
            my_off = off
            rdma = pltpu.make_async_remote_copy(
                src_ref=ag_ref.at[pl.ds(my_off, seg), :],
                dst_ref=ag_ref.at[pl.ds(my_off, seg), :],
                send_sem=send_sems.at[3 + k],
                recv_sem=recv_sems.at[3 + k],
                device_id=(partners[k],),
                device_id_type=pl.DeviceIdType.MESH,
            )
            rdma.start()
            rdma.wait()
            off = my_off - b * seg

        out_ref[0, :, :] = ag_ref[...].astype(jnp.float32)

    out = pl.pallas_call(
        body,
        out_shape=jax.ShapeDtypeStruct((1, SQ, D), jnp.float32),
        in_specs=[pl.BlockSpec(memory_space=pltpu.VMEM)] * 5,
        out_specs=pl.BlockSpec(memory_space=pltpu.VMEM),
        scratch_shapes=[
            pltpu.VMEM((SQ, D), jnp.bfloat16),
            pltpu.VMEM((SQ, D), jnp.bfloat16),
            pltpu.VMEM((256, D), jnp.bfloat16),
            pltpu.VMEM((128, D), jnp.bfloat16),
            pltpu.VMEM((64, D), jnp.bfloat16),
            pltpu.SemaphoreType.DMA((6,)),
            pltpu.SemaphoreType.DMA((6,)),
        ],
        compiler_params=pltpu.CompilerParams(collective_id=0),
    )(x2, Wq, Wo, wk_loc, wv_loc)
    return out


# device time: 38585 ns/iter; 2.1347x vs baseline; 1.2299x over previous
import jax
import jax.numpy as jnp
from jax import lax
from jax.experimental import pallas as pl
from jax.experimental.pallas import tpu as pltpu

N_DEV = 8
SQ = 512
D = 1024
DH = 128
HQ_LOC = 8
KV_LOC = 2
SCALE = 0.08838834764831843

_MASKS = (1, 3, 4)
HALF = 256
_PARTS = ((0, (0, 1, 2)), (HALF, (2, 0, 1)))
_RSBUF_OFF = (((0, 128), (128, 64), (192, 32)),
              ((224, 128), (352, 64), (416, 32)))


def kernel(x, Wq, Wo, Wk, Wv):
    my = lax.axis_index("i")
    x2 = x.reshape(SQ, D)
    wk_loc = lax.dynamic_slice(Wk, (0, my * (KV_LOC * DH)), (D, KV_LOC * DH))
    wv_loc = lax.dynamic_slice(Wv, (0, my * (KV_LOC * DH)), (D, KV_LOC * DH))

    def body(x_ref, wq_ref, wo_ref, wk_ref, wv_ref, out_ref,
             p_ref, ag_ref, rsbuf_ref, send_sems, recv_sems):
        my_i = lax.axis_index("i")
        q4 = my_i % 4
        bz = my_i // 4
        by = q4 // 2
        bx = (q4 % 2) ^ by
        bits = (bx, by, bz)
        partners = tuple((my_i ^ m) for m in _MASKS)

        barrier_sem = pltpu.get_barrier_semaphore()
        for nbr in partners:
            pl.semaphore_signal(
                barrier_sem, inc=1,
                device_id=(nbr,), device_id_type=pl.DeviceIdType.MESH,
            )
        pl.semaphore_wait(barrier_sem, 3)

        xv = x_ref[...].astype(jnp.bfloat16)
        qm = jnp.dot(xv, wq_ref[...].astype(jnp.bfloat16),
                     preferred_element_type=jnp.float32)
        km = jnp.dot(xv, wk_ref[...].astype(jnp.bfloat16),
                     preferred_element_type=jnp.float32)
        vm = jnp.dot(xv, wv_ref[...].astype(jnp.bfloat16),
                     preferred_element_type=jnp.float32)

        o_heads = []
        for h in range(HQ_LOC):
            qh = qm[:, h * DH:(h + 1) * DH].astype(jnp.bfloat16)
            kv = h // 4
            kh = km[:, kv * DH:(kv + 1) * DH].astype(jnp.bfloat16)
            vh = vm[:, kv * DH:(kv + 1) * DH].astype(jnp.bfloat16)
            s = jnp.dot(qh, kh.T, preferred_element_type=jnp.float32) * SCALE
            m = jnp.max(s, axis=-1, keepdims=True)
            p = jnp.exp(s - m).astype(jnp.bfloat16)
            l = jnp.sum(p, axis=-1, keepdims=True, dtype=jnp.float32)
            o_heads.append(
                jnp.dot(p, vh, preferred_element_type=jnp.float32) / l
            )
        o_loc = jnp.concatenate(o_heads, axis=1).astype(jnp.bfloat16)

        p_ref[...] = jnp.dot(
            o_loc, wo_ref[...].astype(jnp.bfloat16),
            preferred_element_type=jnp.float32,
        ).astype(jnp.bfloat16)

        off = [jnp.int32(0), jnp.int32(0)]
        pending = [None, None]

        def rs_start(ip, s):
            base, dims = _PARTS[ip]
            b = bits[dims[s]]
            seg = HALF >> (s + 1)
            send_off = base + off[ip] + (1 - b) * seg
            buf_off, _ = _RSBUF_OFF[ip][s]
            rdma = pltpu.make_async_remote_copy(
                src_ref=p_ref.at[pl.ds(send_off, seg), :],
                dst_ref=rsbuf_ref.at[pl.ds(buf_off, seg), :],
                send_sem=send_sems.at[ip * 6 + s],
                recv_sem=recv_sems.at[ip * 6 + s],
                device_id=(partners[dims[s]],),
                device_id_type=pl.DeviceIdType.MESH,
            )
            rdma.start()
            pending[ip] = ("rs", rdma, s)

        def ag_start(ip, s):
            base, dims = _PARTS[ip]
            seg = 32 << s
            src = ag_ref.at[pl.ds(base + off[ip], seg), :]
            rdma = pltpu.make_async_remote_copy(
                src_ref=src,
                dst_ref=src,
                send_sem=send_sems.at[ip * 6 + 3 + s],
                recv_sem=recv_sems.at[ip * 6 + 3 + s],
                device_id=(partners[dims[2 - s]],),
                device_id_type=pl.DeviceIdType.MESH,
            )
            rdma.start()
            pending[ip] = ("ag", rdma, s)

        def finish(ip):
            if pending[ip] is None:
                return
            kind, rdma, s = pending[ip]
            pending[ip] = None
            base, dims = _PARTS[ip]
            rdma.wait()
            if kind == "rs":
                b = bits[dims[s]]
                seg = HALF >> (s + 1)
                buf_off, _ = _RSBUF_OFF[ip][s]
                keep = base + off[ip] + b * seg
                p_ref[pl.ds(keep, seg), :] = (
                    p_ref[pl.ds(keep, seg), :]
                    + rsbuf_ref[pl.ds(buf_off, seg), :]
                )
                off[ip] = off[ip] + b * seg
                if s == 2:
                    ag_ref[pl.ds(base + off[ip], 32), :] = (
                        p_ref[pl.ds(base + off[ip], 32), :]
                    )
            else:
                b = bits[dims[2 - s]]
                seg = 32 << s
                off[ip] = off[ip] - b * seg

        for step in range(6):
            for ip in range(2):
                finish(ip)
                if step < 3:
                    rs_start(ip, step)
                else:
                    ag_start(ip, step - 3)
        for ip in range(2):
            finish(ip)

        out_ref[0, :, :] = ag_ref[...].astype(jnp.float32)

    out = pl.pallas_call(
        body,
        out_shape=jax.ShapeDtypeStruct((1, SQ, D), jnp.float32),
        in_specs=[pl.BlockSpec(memory_space=pltpu.VMEM)] * 5,
        out_specs=pl.BlockSpec(memory_space=pltpu.VMEM),
        scratch_shapes=[
            pltpu.VMEM((SQ, D), jnp.bfloat16),
            pltpu.VMEM((SQ, D), jnp.bfloat16),
            pltpu.VMEM((448, D), jnp.bfloat16),
            pltpu.SemaphoreType.DMA((12,)),
            pltpu.SemaphoreType.DMA((12,)),
        ],
        compiler_params=pltpu.CompilerParams(collective_id=0),
    )(x2, Wq, Wo, wk_loc, wv_loc)
    return out
